# baseline (speedup 1.0000x reference)
import functools

import jax
import jax.numpy as jnp
from jax import lax
from jax.experimental import pallas as pl
from jax.experimental.pallas import tpu as pltpu
from jax.experimental.pallas import tpu_sc as plsc

B, S, D = 4, 8192, 1024
NC, NS = 2, 16
NW = NC * NS
POS_PER_W = S // NW
CHUNK = 16
NCHUNK = POS_PER_W // CHUNK
CHUNK_ELEMS = CHUNK * D
LANES = 16
NVEC = CHUNK_ELEMS // LANES


def _sc_body(in_hbm, pos_hbm, out_hbm, pos_buf, data_buf):
    wid = lax.axis_index("s") * NC + lax.axis_index("c")
    base_pos = wid * POS_PER_W

    def chunk_body(c, carry):
        pos_off = (base_pos + c * CHUNK) * D
        pltpu.sync_copy(pos_hbm.at[pl.ds(pos_off, CHUNK_ELEMS)], pos_buf)

        def batch_body(b, carry2):
            off = b * (S * D) + pos_off
            pltpu.sync_copy(in_hbm.at[pl.ds(off, CHUNK_ELEMS)], data_buf)

            def add_body(i, carry3):
                sl = pl.ds(i * LANES, LANES)
                data_buf[sl] = data_buf[sl] + pos_buf[sl]
                return carry3

            lax.fori_loop(0, NVEC, add_body, 0)
            pltpu.sync_copy(data_buf, out_hbm.at[pl.ds(off, CHUNK_ELEMS)])
            return carry2

        lax.fori_loop(0, B, batch_body, 0)
        return carry

    lax.fori_loop(0, NCHUNK, chunk_body, 0)


_sc_add = functools.partial(
    pl.kernel,
    mesh=plsc.VectorSubcoreMesh(core_axis_name="c", subcore_axis_name="s"),
    out_type=jax.ShapeDtypeStruct((B * S * D,), jnp.float32),
    scratch_types=[
        pltpu.VMEM((CHUNK_ELEMS,), jnp.float32),
        pltpu.VMEM((CHUNK_ELEMS,), jnp.float32),
    ],
)(_sc_body)


@jax.jit
def kernel(inputs, pos_table):
    in_flat = inputs.reshape(B * S * D)
    pos_flat = pos_table.reshape(S * D)
    out = _sc_add(in_flat, pos_flat)
    return out.reshape(B, S, D)
